# baseline (speedup 1.0000x reference)
import functools

import jax
import jax.numpy as jnp
from jax import lax
from jax.experimental import pallas as pl
from jax.experimental.pallas import tpu as pltpu
from jax.experimental.pallas import tpu_sc as plsc

N = 10000
NP = 10240
D = 128
E = 320000
NSC = 2
NSUB = 16
CHUNK = 80
RPS = NP // NSUB
BR = 1024

_sc_mesh = functools.partial(
    plsc.VectorSubcoreMesh, core_axis_name="c", subcore_axis_name="s")


def _seg_pipeline(nchunk, gath, gath_wait, scat, scat_wait,
                  idx_load, idx_ready):
  def step(j, b, first=False):
    gath_wait(j, b)
    scat(j, b)
    if not first:
      scat_wait(j - 1, (b + 3) % 4)

    @pl.when(j + 3 < nchunk)
    def _():
      idx_ready(j + 3, (b + 3) % 4)
      gath(j + 3, (b + 3) % 4)

    @pl.when(j + 4 < nchunk)
    def _():
      idx_load(j + 4, b % 2)

  idx_load(0, 0)
  idx_load(1, 1)
  idx_ready(0, 0)
  gath(0, 0)
  idx_load(2, 0)
  idx_ready(1, 1)
  gath(1, 1)
  idx_load(3, 1)
  idx_ready(2, 2)
  gath(2, 2)

  step(0, 0, first=True)
  for j in (1, 2, 3):
    step(j, j)

  nquads = nchunk // 4

  @pl.loop(1, nquads)
  def _(i):
    step(4 * i, 0)
    step(4 * i + 1, 1)
    step(4 * i + 2, 2)
    step(4 * i + 3, 3)

  for j in range(4 * nquads, nchunk):
    step(j, j % 4)
  scat_wait(nchunk - 1, (nchunk - 1) % 4)


def _seg_scratch(tail):
  return [
      pltpu.VMEM_SHARED((NP, D), jnp.float32),
      pltpu.VMEM_SHARED((NP,), jnp.float32),
      [pltpu.VMEM((CHUNK,), jnp.int32)] * 2,
      [pltpu.VMEM((CHUNK,), jnp.int32)] * 2,
      [pltpu.VMEM((CHUNK,), jnp.int32)] * 4,
      [pltpu.VMEM((CHUNK,), jnp.int32)] * 4,
      [pltpu.VMEM((CHUNK, D), jnp.float32)] * 4,
      pltpu.VMEM((CHUNK,), jnp.float32),
      pltpu.VMEM((tail,), jnp.int32),
      pltpu.VMEM((tail,), jnp.int32),
      pltpu.VMEM((tail, D), jnp.float32),
      pltpu.VMEM((tail,), jnp.float32),
      [pltpu.SemaphoreType.DMA] * 4,
      [pltpu.SemaphoreType.DMA] * 4,
      [pltpu.SemaphoreType.DMA] * 2,
  ]


def _seg_sum_split_edges(table, edges_flat, z2, z1):
  ept = E // (NSC * NSUB)
  nchunk = ept // CHUNK
  tail = ept - nchunk * CHUNK

  @functools.partial(
      pl.kernel,
      out_type=(jax.ShapeDtypeStruct((NSC, NP, D), jnp.float32),
                jax.ShapeDtypeStruct((NSC, NP), jnp.float32)),
      mesh=_sc_mesh(),
      scratch_types=_seg_scratch(tail),
  )
  def k(table_hbm, ei_hbm, z2_hbm, z1_hbm, agg_hbm, cnt_hbm,
        acc_sh, cnt_sh, psrc_v, pdst_v, src_v, dst_v, rows_v, ones_v,
        tsrc_v, tdst_v, trows_v, tones_v, g_s, s_s, i_s):
    cid = lax.axis_index("c")
    sid = lax.axis_index("s")
    r0 = sid * RPS
    pltpu.sync_copy(z2_hbm.at[pl.ds(r0, RPS)], acc_sh.at[pl.ds(r0, RPS)])
    pltpu.sync_copy(z1_hbm.at[pl.ds(r0, RPS)], cnt_sh.at[pl.ds(r0, RPS)])
    for i in range(CHUNK // 16):
      ones_v[pl.ds(i * 16, 16)] = jnp.ones((16,), jnp.float32)
    for i in range(tail // 16):
      tones_v[pl.ds(i * 16, 16)] = jnp.ones((16,), jnp.float32)
    plsc.subcore_barrier()

    base = (cid * NSUB + sid) * ept

    def idx_load(j, p):
      off = base + j * CHUNK
      pltpu.async_copy(ei_hbm.at[pl.ds(off, CHUNK)], psrc_v[p], i_s[p])
      pltpu.async_copy(ei_hbm.at[pl.ds(E + off, CHUNK)], pdst_v[p], i_s[p])

    def idx_ready(j, b):
      p = b % 2
      off = base + j * CHUNK
      pltpu.make_async_copy(ei_hbm.at[pl.ds(off, CHUNK)], psrc_v[p],
                            i_s[p]).wait()
      pltpu.make_async_copy(ei_hbm.at[pl.ds(E + off, CHUNK)], pdst_v[p],
                            i_s[p]).wait()
      for i in range(CHUNK // 16):
        sl = pl.ds(i * 16, 16)
        src_v[b][sl] = psrc_v[p][sl]
        dst_v[b][sl] = pdst_v[p][sl]

    def gath(j, x):
      pltpu.async_copy(table_hbm.at[src_v[x]], rows_v[x], g_s[x])

    def gath_wait(j, x):
      pltpu.make_async_copy(table_hbm.at[src_v[x]], rows_v[x],
                            g_s[x]).wait()

    def scat(j, x):
      pltpu.async_copy(rows_v[x], acc_sh.at[dst_v[x]], s_s[x], add=True)
      pltpu.async_copy(ones_v, cnt_sh.at[dst_v[x]], s_s[x], add=True)

    def scat_wait(j, x):
      pltpu.make_async_copy(rows_v[x], acc_sh.at[dst_v[x]], s_s[x]).wait()
      pltpu.make_async_copy(ones_v, cnt_sh.at[dst_v[x]], s_s[x]).wait()

    _seg_pipeline(nchunk, gath, gath_wait, scat, scat_wait,
                  idx_load, idx_ready)

    if tail:
      toff = base + nchunk * CHUNK
      pltpu.sync_copy(ei_hbm.at[pl.ds(toff, tail)], tsrc_v)
      pltpu.sync_copy(ei_hbm.at[pl.ds(E + toff, tail)], tdst_v)
      pltpu.sync_copy(table_hbm.at[tsrc_v], trows_v)
      pltpu.sync_copy(trows_v, acc_sh.at[tdst_v], add=True)
      pltpu.sync_copy(tones_v, cnt_sh.at[tdst_v], add=True)

    plsc.subcore_barrier()
    pltpu.sync_copy(acc_sh.at[pl.ds(r0, RPS)], agg_hbm.at[cid, pl.ds(r0, RPS)])
    pltpu.sync_copy(cnt_sh.at[pl.ds(r0, RPS)], cnt_hbm.at[cid, pl.ds(r0, RPS)])

  return k(table, edges_flat, z2, z1)


def _seg_sum_two_tables(tables_flat, edges_flat, z2, z1):
  ept = E // NSUB
  nchunk = ept // CHUNK
  tail = ept - nchunk * CHUNK

  @functools.partial(
      pl.kernel,
      out_type=(jax.ShapeDtypeStruct((NSC, NP, D), jnp.float32),
                jax.ShapeDtypeStruct((NP,), jnp.float32)),
      mesh=_sc_mesh(),
      scratch_types=_seg_scratch(tail),
  )
  def k(tab_hbm, ei_hbm, z2_hbm, z1_hbm, agg_hbm, cnt_hbm,
        acc_sh, cnt_sh, psrc_v, pdst_v, src_v, dst_v, rows_v, ones_v,
        tsrc_v, tdst_v, trows_v, tones_v, g_s, s_s, i_s):
    cid = lax.axis_index("c")
    sid = lax.axis_index("s")
    r0 = sid * RPS
    row_off = cid * NP
    pltpu.sync_copy(z2_hbm.at[pl.ds(r0, RPS)], acc_sh.at[pl.ds(r0, RPS)])

    @pl.when(cid == 0)
    def _():
      pltpu.sync_copy(z1_hbm.at[pl.ds(r0, RPS)], cnt_sh.at[pl.ds(r0, RPS)])

    for i in range(CHUNK // 16):
      ones_v[pl.ds(i * 16, 16)] = jnp.ones((16,), jnp.float32)
    for i in range(tail // 16):
      tones_v[pl.ds(i * 16, 16)] = jnp.ones((16,), jnp.float32)
    plsc.subcore_barrier()

    base = sid * ept

    def idx_load(j, p):
      off = base + j * CHUNK
      pltpu.async_copy(ei_hbm.at[pl.ds(off, CHUNK)], psrc_v[p], i_s[p])
      pltpu.async_copy(ei_hbm.at[pl.ds(E + off, CHUNK)], pdst_v[p], i_s[p])

    def idx_ready(j, b):
      p = b % 2
      off = base + j * CHUNK
      pltpu.make_async_copy(ei_hbm.at[pl.ds(off, CHUNK)], psrc_v[p],
                            i_s[p]).wait()
      pltpu.make_async_copy(ei_hbm.at[pl.ds(E + off, CHUNK)], pdst_v[p],
                            i_s[p]).wait()
      for i in range(CHUNK // 16):
        sl = pl.ds(i * 16, 16)
        src_v[b][sl] = psrc_v[p][sl] + row_off
        dst_v[b][sl] = pdst_v[p][sl]

    def gath(j, x):
      pltpu.async_copy(tab_hbm.at[src_v[x]], rows_v[x], g_s[x])

    def gath_wait(j, x):
      pltpu.make_async_copy(tab_hbm.at[src_v[x]], rows_v[x],
                            g_s[x]).wait()

    def scat(j, x):
      pltpu.async_copy(rows_v[x], acc_sh.at[dst_v[x]], s_s[x], add=True)

      @pl.when(cid == 0)
      def _():
        pltpu.async_copy(ones_v, cnt_sh.at[dst_v[x]], s_s[x], add=True)

    def scat_wait(j, x):
      pltpu.make_async_copy(rows_v[x], acc_sh.at[dst_v[x]], s_s[x]).wait()

      @pl.when(cid == 0)
      def _():
        pltpu.make_async_copy(ones_v, cnt_sh.at[dst_v[x]], s_s[x]).wait()

    _seg_pipeline(nchunk, gath, gath_wait, scat, scat_wait,
                  idx_load, idx_ready)

    if tail:
      toff = base + nchunk * CHUNK
      pltpu.sync_copy(ei_hbm.at[pl.ds(toff, tail)], tsrc_v)
      pltpu.sync_copy(ei_hbm.at[pl.ds(E + toff, tail)], tdst_v)
      for i in range(tail // 16):
        sl = pl.ds(i * 16, 16)
        tsrc_v[sl] = tsrc_v[sl] + row_off
      pltpu.sync_copy(tab_hbm.at[tsrc_v], trows_v)
      pltpu.sync_copy(trows_v, acc_sh.at[tdst_v], add=True)

      @pl.when(cid == 0)
      def _():
        pltpu.sync_copy(tones_v, cnt_sh.at[tdst_v], add=True)

    plsc.subcore_barrier()
    pltpu.sync_copy(acc_sh.at[pl.ds(r0, RPS)], agg_hbm.at[cid, pl.ds(r0, RPS)])

    @pl.when(cid == 0)
    def _():
      pltpu.sync_copy(cnt_sh.at[pl.ds(r0, RPS)], cnt_hbm.at[pl.ds(r0, RPS)])

  return k(tables_flat, edges_flat, z2, z1)


def _x_side(x, w, b):
  cols = w.shape[1]

  def body(x_ref, w_ref, b_ref, o_ref):
    o_ref[...] = (jnp.dot(x_ref[...], w_ref[...],
                          preferred_element_type=jnp.float32) + b_ref[0])

  return pl.pallas_call(
      body,
      grid=(NP // BR,),
      in_specs=[
          pl.BlockSpec((BR, D), lambda i: (i, 0)),
          pl.BlockSpec((D, cols), lambda i: (0, 0)),
          pl.BlockSpec((1, cols), lambda i: (0, 0)),
      ],
      out_specs=pl.BlockSpec((BR, cols), lambda i: (i, 0)),
      out_shape=jax.ShapeDtypeStruct((NP, cols), jnp.float32),
  )(x, w, b)


def _hop1_combine(aggP, cntP, pre, wl, bu):
  def body(a_ref, c_ref, p_ref, wl_ref, bu_ref, o_ref):
    agg = a_ref[0] + a_ref[1]
    cnt = c_ref[0] + c_ref[1]
    inv = 1.0 / jnp.maximum(cnt, 1.0)
    mask = jnp.where(cnt > 0.0, 1.0, 0.0)
    z = (jnp.dot(agg, wl_ref[...], preferred_element_type=jnp.float32)
         * inv[:, None]
         + mask[:, None] * bu_ref[0]
         + p_ref[...])
    h = jnp.maximum(z, 0.0)
    o_ref[0] = h[:, :D]
    o_ref[1] = h[:, D:]

  return pl.pallas_call(
      body,
      grid=(NP // BR,),
      in_specs=[
          pl.BlockSpec((2, BR, D), lambda i: (0, i, 0)),
          pl.BlockSpec((2, BR), lambda i: (0, i)),
          pl.BlockSpec((BR, 2 * D), lambda i: (i, 0)),
          pl.BlockSpec((D, 2 * D), lambda i: (0, 0)),
          pl.BlockSpec((1, 2 * D), lambda i: (0, 0)),
      ],
      out_specs=pl.BlockSpec((2, BR, D), lambda i: (0, i, 0)),
      out_shape=jax.ShapeDtypeStruct((2, NP, D), jnp.float32),
  )(aggP, cntP, pre, wl, bu)


def _hop2_final(agg2, cnt2, pre, wlA, wlB, v0, v1, c):
  def body(c_s, a_ref, c2_ref, p_ref, wla, wlb, v0_ref, v1_ref, o_ref):
    inv = 1.0 / jnp.maximum(c2_ref[...], 1.0)
    zA = (jnp.dot(a_ref[0], wla[...], preferred_element_type=jnp.float32)
          * inv[:, None] + p_ref[:, :D])
    zB = (jnp.dot(a_ref[1], wlb[...], preferred_element_type=jnp.float32)
          * inv[:, None] + p_ref[:, D:])
    hA = jnp.maximum(zA, 0.0)
    hB = jnp.maximum(zB, 0.0)
    y = (jnp.sum(hA * v0_ref[0], axis=1)
         + jnp.sum(hB * v1_ref[0], axis=1) + c_s[0])
    o_ref[0, 0, :] = y

  return pl.pallas_call(
      body,
      grid=(NP // BR,),
      in_specs=[
          pl.BlockSpec(memory_space=pltpu.SMEM),
          pl.BlockSpec((2, BR, D), lambda i: (0, i, 0)),
          pl.BlockSpec((BR,), lambda i: (i,)),
          pl.BlockSpec((BR, 2 * D), lambda i: (i, 0)),
          pl.BlockSpec((D, D), lambda i: (0, 0)),
          pl.BlockSpec((D, D), lambda i: (0, 0)),
          pl.BlockSpec((1, D), lambda i: (0, 0)),
          pl.BlockSpec((1, D), lambda i: (0, 0)),
      ],
      out_specs=pl.BlockSpec((1, 1, BR), lambda i: (i, 0, 0)),
      out_shape=jax.ShapeDtypeStruct((NP // BR, 1, BR), jnp.float32),
  )(c, agg2, cnt2, pre, wlA, wlB, v0, v1)


def kernel(x_user, x_item, edge_u2i, edge_i2u, W_enc_u, b_enc_u, W_enc_i,
           b_enc_i, Wl00, bl00, Wr00, Wl01, bl01, Wr01, Wp0, bp0, Wl10,
           bl10, Wr10, Wl11, bl11, Wr11, Wp1, bp1, W_reg, b_reg):
  f32 = jnp.float32

  wl1 = jnp.concatenate([Wl00.T, Wl10.T], axis=1)
  wr1 = jnp.concatenate([Wr00.T, Wr10.T], axis=1)
  bl1 = jnp.concatenate([bl00, bl10])
  wlB = W_enc_u.T @ wl1
  buB = (b_enc_u @ wl1)[None, :]
  wrB = W_enc_i.T @ wr1
  bB = (b_enc_i @ wr1 + bl1)[None, :]
  wlCA = Wl01.T
  wrCA = W_enc_u.T @ Wr01.T
  bCA = (b_enc_u @ Wr01.T + bl01)[None, :]
  buCA = jnp.zeros((1, D), f32)
  wlCB = Wl11.T
  wrCB = W_enc_u.T @ Wr11.T
  bCB = (b_enc_u @ Wr11.T + bl11)[None, :]
  buCB = jnp.zeros((1, D), f32)
  wreg = W_reg[0]
  v0 = (Wp0.T @ wreg[:64])[None, :]
  v1 = (Wp1.T @ wreg[64:])[None, :]
  c = (jnp.dot(bp0, wreg[:64]) + jnp.dot(bp1, wreg[64:])
       + b_reg[0]).reshape(1).astype(f32)

  z2 = jnp.zeros((NP, D), f32)
  z1 = jnp.zeros((NP,), f32)

  agg1P, cnt1P = _seg_sum_split_edges(x_user, edge_u2i.reshape(2 * E),
                                      z2, z1)
  preB = _x_side(x_item, wrB, bB)
  h01 = _hop1_combine(agg1P, cnt1P, preB, wlB, buB)
  agg2, cnt2 = _seg_sum_two_tables(h01.reshape(NSC * NP, D),
                                   edge_i2u.reshape(2 * E), z2, z1)
  wrC = jnp.concatenate([wrCA, wrCB], axis=1)
  bC = jnp.concatenate([bCA[0], bCB[0]])[None, :]
  preC = _x_side(x_user, wrC, bC)
  y = _hop2_final(agg2, cnt2, preC, wlCA, wlCB, v0, v1, c)
  return y.reshape(NP)[:N]


# speedup vs baseline: 1.0136x; 1.0136x over previous
import functools

import jax
import jax.numpy as jnp
from jax import lax
from jax.experimental import pallas as pl
from jax.experimental.pallas import tpu as pltpu
from jax.experimental.pallas import tpu_sc as plsc

N = 10000
NP = 10240
D = 128
E = 320000
NSC = 2
NSUB = 16
CHUNK = 80
RPS = NP // NSUB
BR = 1024

_sc_mesh = functools.partial(
    plsc.VectorSubcoreMesh, core_axis_name="c", subcore_axis_name="s")


def _seg_pipeline(nchunk, gath, gath_wait, scat, scat_wait,
                  idx_load, idx_ready):
  def step(j, b, first=False):
    gath_wait(j, b)
    scat(j, b)
    if not first:
      scat_wait(j - 1, (b + 3) % 4)

    @pl.when(j + 3 < nchunk)
    def _():
      idx_ready(j + 3, (b + 3) % 4)
      gath(j + 3, (b + 3) % 4)

    @pl.when(j + 4 < nchunk)
    def _():
      idx_load(j + 4, b % 2)

  idx_load(0, 0)
  idx_load(1, 1)
  idx_ready(0, 0)
  gath(0, 0)
  idx_load(2, 0)
  idx_ready(1, 1)
  gath(1, 1)
  idx_load(3, 1)
  idx_ready(2, 2)
  gath(2, 2)

  step(0, 0, first=True)
  for j in (1, 2, 3):
    step(j, j)

  nquads = nchunk // 4

  @pl.loop(1, nquads)
  def _(i):
    step(4 * i, 0)
    step(4 * i + 1, 1)
    step(4 * i + 2, 2)
    step(4 * i + 3, 3)

  for j in range(4 * nquads, nchunk):
    step(j, j % 4)
  scat_wait(nchunk - 1, (nchunk - 1) % 4)


def _seg_scratch(tail):
  return [
      pltpu.VMEM_SHARED((NP, D), jnp.float32),
      pltpu.VMEM_SHARED((NP,), jnp.float32),
      [pltpu.VMEM((CHUNK,), jnp.int32)] * 2,
      [pltpu.VMEM((CHUNK,), jnp.int32)] * 2,
      [pltpu.VMEM((CHUNK,), jnp.int32)] * 4,
      [pltpu.VMEM((CHUNK,), jnp.int32)] * 4,
      [pltpu.VMEM((CHUNK, D), jnp.float32)] * 4,
      pltpu.VMEM((CHUNK,), jnp.float32),
      pltpu.VMEM((tail,), jnp.int32),
      pltpu.VMEM((tail,), jnp.int32),
      pltpu.VMEM((tail, D), jnp.float32),
      pltpu.VMEM((tail,), jnp.float32),
      [pltpu.SemaphoreType.DMA] * 4,
      [pltpu.SemaphoreType.DMA] * 4,
      [pltpu.SemaphoreType.DMA] * 2,
  ]


def _seg_sum_split_edges(table, edges_flat, z2, z1):
  ept = E // (NSC * NSUB)
  nchunk = ept // CHUNK
  tail = ept - nchunk * CHUNK

  @functools.partial(
      pl.kernel,
      out_type=(jax.ShapeDtypeStruct((NSC, NP, D), jnp.float32),
                jax.ShapeDtypeStruct((NSC, NP), jnp.float32)),
      mesh=_sc_mesh(),
      scratch_types=_seg_scratch(tail),
  )
  def k(table_hbm, ei_hbm, z2_hbm, z1_hbm, agg_hbm, cnt_hbm,
        acc_sh, cnt_sh, psrc_v, pdst_v, src_v, dst_v, rows_v, ones_v,
        tsrc_v, tdst_v, trows_v, tones_v, g_s, s_s, i_s):
    cid = lax.axis_index("c")
    sid = lax.axis_index("s")
    r0 = sid * RPS
    pltpu.sync_copy(z2_hbm.at[pl.ds(r0, RPS)], acc_sh.at[pl.ds(r0, RPS)])
    pltpu.sync_copy(z1_hbm.at[pl.ds(r0, RPS)], cnt_sh.at[pl.ds(r0, RPS)])
    for i in range(CHUNK // 16):
      ones_v[pl.ds(i * 16, 16)] = jnp.ones((16,), jnp.float32)
    for i in range(tail // 16):
      tones_v[pl.ds(i * 16, 16)] = jnp.ones((16,), jnp.float32)
    plsc.subcore_barrier()

    base = (cid * NSUB + sid) * ept

    def idx_load(j, p):
      off = base + j * CHUNK
      pltpu.async_copy(ei_hbm.at[pl.ds(off, CHUNK)], psrc_v[p], i_s[p])
      pltpu.async_copy(ei_hbm.at[pl.ds(E + off, CHUNK)], pdst_v[p], i_s[p])

    def idx_ready(j, b):
      p = b % 2
      off = base + j * CHUNK
      pltpu.make_async_copy(ei_hbm.at[pl.ds(off, CHUNK)], psrc_v[p],
                            i_s[p]).wait()
      pltpu.make_async_copy(ei_hbm.at[pl.ds(E + off, CHUNK)], pdst_v[p],
                            i_s[p]).wait()
      for i in range(CHUNK // 16):
        sl = pl.ds(i * 16, 16)
        src_v[b][sl] = psrc_v[p][sl]
        dst_v[b][sl] = pdst_v[p][sl]

    def gath(j, x):
      pltpu.async_copy(table_hbm.at[src_v[x]], rows_v[x], g_s[x])

    def gath_wait(j, x):
      pltpu.make_async_copy(table_hbm.at[src_v[x]], rows_v[x],
                            g_s[x]).wait()

    def scat(j, x):
      pltpu.async_copy(rows_v[x], acc_sh.at[dst_v[x]], s_s[x], add=True)
      pltpu.async_copy(ones_v, cnt_sh.at[dst_v[x]], s_s[x], add=True)

    def scat_wait(j, x):
      pltpu.make_async_copy(rows_v[x], acc_sh.at[dst_v[x]], s_s[x]).wait()
      pltpu.make_async_copy(ones_v, cnt_sh.at[dst_v[x]], s_s[x]).wait()

    _seg_pipeline(nchunk, gath, gath_wait, scat, scat_wait,
                  idx_load, idx_ready)

    if tail:
      toff = base + nchunk * CHUNK
      pltpu.sync_copy(ei_hbm.at[pl.ds(toff, tail)], tsrc_v)
      pltpu.sync_copy(ei_hbm.at[pl.ds(E + toff, tail)], tdst_v)
      pltpu.sync_copy(table_hbm.at[tsrc_v], trows_v)
      pltpu.sync_copy(trows_v, acc_sh.at[tdst_v], add=True)
      pltpu.sync_copy(tones_v, cnt_sh.at[tdst_v], add=True)

    plsc.subcore_barrier()
    pltpu.sync_copy(acc_sh.at[pl.ds(r0, RPS)], agg_hbm.at[cid, pl.ds(r0, RPS)])
    pltpu.sync_copy(cnt_sh.at[pl.ds(r0, RPS)], cnt_hbm.at[cid, pl.ds(r0, RPS)])

  return k(table, edges_flat, z2, z1)


def _seg_sum_two_tables(tables_flat, edges_flat, z2, z1):
  ept = E // NSUB
  nchunk = ept // CHUNK
  tail = ept - nchunk * CHUNK

  @functools.partial(
      pl.kernel,
      out_type=(jax.ShapeDtypeStruct((NSC, NP, D), jnp.float32),
                jax.ShapeDtypeStruct((NP,), jnp.float32)),
      mesh=_sc_mesh(),
      scratch_types=_seg_scratch(tail),
  )
  def k(tab_hbm, ei_hbm, z2_hbm, z1_hbm, agg_hbm, cnt_hbm,
        acc_sh, cnt_sh, psrc_v, pdst_v, src_v, dst_v, rows_v, ones_v,
        tsrc_v, tdst_v, trows_v, tones_v, g_s, s_s, i_s):
    cid = lax.axis_index("c")
    sid = lax.axis_index("s")
    r0 = sid * RPS
    row_off = cid * NP
    pltpu.sync_copy(z2_hbm.at[pl.ds(r0, RPS)], acc_sh.at[pl.ds(r0, RPS)])

    @pl.when(cid == 0)
    def _():
      pltpu.sync_copy(z1_hbm.at[pl.ds(r0, RPS)], cnt_sh.at[pl.ds(r0, RPS)])

    for i in range(CHUNK // 16):
      ones_v[pl.ds(i * 16, 16)] = jnp.ones((16,), jnp.float32)
    for i in range(tail // 16):
      tones_v[pl.ds(i * 16, 16)] = jnp.ones((16,), jnp.float32)
    plsc.subcore_barrier()

    base = sid * ept

    def idx_load(j, p):
      off = base + j * CHUNK
      pltpu.async_copy(ei_hbm.at[pl.ds(off, CHUNK)], psrc_v[p], i_s[p])
      pltpu.async_copy(ei_hbm.at[pl.ds(E + off, CHUNK)], pdst_v[p], i_s[p])

    def idx_ready(j, b):
      p = b % 2
      off = base + j * CHUNK
      pltpu.make_async_copy(ei_hbm.at[pl.ds(off, CHUNK)], psrc_v[p],
                            i_s[p]).wait()
      pltpu.make_async_copy(ei_hbm.at[pl.ds(E + off, CHUNK)], pdst_v[p],
                            i_s[p]).wait()
      for i in range(CHUNK // 16):
        sl = pl.ds(i * 16, 16)
        src_v[b][sl] = psrc_v[p][sl] + row_off
        dst_v[b][sl] = pdst_v[p][sl]

    def gath(j, x):
      pltpu.async_copy(tab_hbm.at[src_v[x]], rows_v[x], g_s[x])

    def gath_wait(j, x):
      pltpu.make_async_copy(tab_hbm.at[src_v[x]], rows_v[x],
                            g_s[x]).wait()

    def scat(j, x):
      pltpu.async_copy(rows_v[x], acc_sh.at[dst_v[x]], s_s[x], add=True)

      @pl.when(cid == 0)
      def _():
        pltpu.async_copy(ones_v, cnt_sh.at[dst_v[x]], s_s[x], add=True)

    def scat_wait(j, x):
      pltpu.make_async_copy(rows_v[x], acc_sh.at[dst_v[x]], s_s[x]).wait()

      @pl.when(cid == 0)
      def _():
        pltpu.make_async_copy(ones_v, cnt_sh.at[dst_v[x]], s_s[x]).wait()

    _seg_pipeline(nchunk, gath, gath_wait, scat, scat_wait,
                  idx_load, idx_ready)

    if tail:
      toff = base + nchunk * CHUNK
      pltpu.sync_copy(ei_hbm.at[pl.ds(toff, tail)], tsrc_v)
      pltpu.sync_copy(ei_hbm.at[pl.ds(E + toff, tail)], tdst_v)
      for i in range(tail // 16):
        sl = pl.ds(i * 16, 16)
        tsrc_v[sl] = tsrc_v[sl] + row_off
      pltpu.sync_copy(tab_hbm.at[tsrc_v], trows_v)
      pltpu.sync_copy(trows_v, acc_sh.at[tdst_v], add=True)

      @pl.when(cid == 0)
      def _():
        pltpu.sync_copy(tones_v, cnt_sh.at[tdst_v], add=True)

    plsc.subcore_barrier()
    pltpu.sync_copy(acc_sh.at[pl.ds(r0, RPS)], agg_hbm.at[cid, pl.ds(r0, RPS)])

    @pl.when(cid == 0)
    def _():
      pltpu.sync_copy(cnt_sh.at[pl.ds(r0, RPS)], cnt_hbm.at[pl.ds(r0, RPS)])

  return k(tables_flat, edges_flat, z2, z1)


def _hop1_combine(aggP, cntP, x_item, wl, wr, b, bu):
  def body(a_ref, c_ref, xi_ref, wl_ref, wr_ref, b_ref, bu_ref, o_ref):
    agg = a_ref[0] + a_ref[1]
    cnt = c_ref[0] + c_ref[1]
    inv = 1.0 / jnp.maximum(cnt, 1.0)
    mask = jnp.where(cnt > 0.0, 1.0, 0.0)
    z = (jnp.dot(agg, wl_ref[...], preferred_element_type=jnp.float32)
         * inv[:, None]
         + mask[:, None] * bu_ref[0]
         + jnp.dot(xi_ref[...], wr_ref[...],
                   preferred_element_type=jnp.float32)
         + b_ref[0])
    h = jnp.maximum(z, 0.0)
    o_ref[0] = h[:, :D]
    o_ref[1] = h[:, D:]

  return pl.pallas_call(
      body,
      grid=(NP // BR,),
      in_specs=[
          pl.BlockSpec((2, BR, D), lambda i: (0, i, 0)),
          pl.BlockSpec((2, BR), lambda i: (0, i)),
          pl.BlockSpec((BR, D), lambda i: (i, 0)),
          pl.BlockSpec((D, 2 * D), lambda i: (0, 0)),
          pl.BlockSpec((D, 2 * D), lambda i: (0, 0)),
          pl.BlockSpec((1, 2 * D), lambda i: (0, 0)),
          pl.BlockSpec((1, 2 * D), lambda i: (0, 0)),
      ],
      out_specs=pl.BlockSpec((2, BR, D), lambda i: (0, i, 0)),
      out_shape=jax.ShapeDtypeStruct((2, NP, D), jnp.float32),
  )(aggP, cntP, x_item, wl, wr, b, bu)


def _hop2_final(agg2, cnt2, x_user, wlA, wrA, bA, wlB, wrB, bB, v0, v1, c):
  def body(c_s, a_ref, c2_ref, xu_ref, wla, wra, ba, wlb, wrb, bb,
           v0_ref, v1_ref, o_ref):
    inv = 1.0 / jnp.maximum(c2_ref[...], 1.0)
    xu_b = xu_ref[...]
    zA = (jnp.dot(a_ref[0], wla[...], preferred_element_type=jnp.float32)
          * inv[:, None]
          + jnp.dot(xu_b, wra[...], preferred_element_type=jnp.float32)
          + ba[0])
    zB = (jnp.dot(a_ref[1], wlb[...], preferred_element_type=jnp.float32)
          * inv[:, None]
          + jnp.dot(xu_b, wrb[...], preferred_element_type=jnp.float32)
          + bb[0])
    hA = jnp.maximum(zA, 0.0)
    hB = jnp.maximum(zB, 0.0)
    y = (jnp.sum(hA * v0_ref[0], axis=1)
         + jnp.sum(hB * v1_ref[0], axis=1) + c_s[0])
    o_ref[0, 0, :] = y

  return pl.pallas_call(
      body,
      grid=(NP // BR,),
      in_specs=[
          pl.BlockSpec(memory_space=pltpu.SMEM),
          pl.BlockSpec((2, BR, D), lambda i: (0, i, 0)),
          pl.BlockSpec((BR,), lambda i: (i,)),
          pl.BlockSpec((BR, D), lambda i: (i, 0)),
          pl.BlockSpec((D, D), lambda i: (0, 0)),
          pl.BlockSpec((D, D), lambda i: (0, 0)),
          pl.BlockSpec((1, D), lambda i: (0, 0)),
          pl.BlockSpec((D, D), lambda i: (0, 0)),
          pl.BlockSpec((D, D), lambda i: (0, 0)),
          pl.BlockSpec((1, D), lambda i: (0, 0)),
          pl.BlockSpec((1, D), lambda i: (0, 0)),
          pl.BlockSpec((1, D), lambda i: (0, 0)),
      ],
      out_specs=pl.BlockSpec((1, 1, BR), lambda i: (i, 0, 0)),
      out_shape=jax.ShapeDtypeStruct((NP // BR, 1, BR), jnp.float32),
  )(c, agg2, cnt2, x_user, wlA, wrA, bA, wlB, wrB, bB, v0, v1)


def kernel(x_user, x_item, edge_u2i, edge_i2u, W_enc_u, b_enc_u, W_enc_i,
           b_enc_i, Wl00, bl00, Wr00, Wl01, bl01, Wr01, Wp0, bp0, Wl10,
           bl10, Wr10, Wl11, bl11, Wr11, Wp1, bp1, W_reg, b_reg):
  f32 = jnp.float32

  wl1 = jnp.concatenate([Wl00.T, Wl10.T], axis=1)
  wr1 = jnp.concatenate([Wr00.T, Wr10.T], axis=1)
  bl1 = jnp.concatenate([bl00, bl10])
  wlB = W_enc_u.T @ wl1
  buB = (b_enc_u @ wl1)[None, :]
  wrB = W_enc_i.T @ wr1
  bB = (b_enc_i @ wr1 + bl1)[None, :]
  wlCA = Wl01.T
  wrCA = W_enc_u.T @ Wr01.T
  bCA = (b_enc_u @ Wr01.T + bl01)[None, :]
  buCA = jnp.zeros((1, D), f32)
  wlCB = Wl11.T
  wrCB = W_enc_u.T @ Wr11.T
  bCB = (b_enc_u @ Wr11.T + bl11)[None, :]
  buCB = jnp.zeros((1, D), f32)
  wreg = W_reg[0]
  v0 = (Wp0.T @ wreg[:64])[None, :]
  v1 = (Wp1.T @ wreg[64:])[None, :]
  c = (jnp.dot(bp0, wreg[:64]) + jnp.dot(bp1, wreg[64:])
       + b_reg[0]).reshape(1).astype(f32)

  z2 = jnp.zeros((NP, D), f32)
  z1 = jnp.zeros((NP,), f32)

  agg1P, cnt1P = _seg_sum_split_edges(x_user, edge_u2i.reshape(2 * E),
                                      z2, z1)
  h01 = _hop1_combine(agg1P, cnt1P, x_item, wlB, wrB, bB, buB)
  agg2, cnt2 = _seg_sum_two_tables(h01.reshape(NSC * NP, D),
                                   edge_i2u.reshape(2 * E), z2, z1)
  y = _hop2_final(agg2, cnt2, x_user, wlCA, wrCA, bCA, wlCB, wrCB, bCB,
                  v0, v1, c)
  return y.reshape(NP)[:N]


# speedup vs baseline: 1.0152x; 1.0016x over previous
import functools

import jax
import jax.numpy as jnp
from jax import lax
from jax.experimental import pallas as pl
from jax.experimental.pallas import tpu as pltpu
from jax.experimental.pallas import tpu_sc as plsc

N = 10000
NP = 10240
D = 128
E = 320000
NSC = 2
NSUB = 16
CHUNK = 80
RPS = NP // NSUB
BR = 1024

_sc_mesh = functools.partial(
    plsc.VectorSubcoreMesh, core_axis_name="c", subcore_axis_name="s")


def _seg_pipeline(nchunk, gath, gath_wait, scat, scat_wait,
                  idx_load, idx_ready):
  def step(j, b, first=False):
    gath_wait(j, b)
    scat(j, b)
    if not first:
      scat_wait(j - 1, (b + 3) % 4)

    @pl.when(j + 3 < nchunk)
    def _():
      idx_ready(j + 3, (b + 3) % 4)
      gath(j + 3, (b + 3) % 4)

    @pl.when(j + 4 < nchunk)
    def _():
      idx_load(j + 4, b % 2)

  idx_load(0, 0)
  idx_load(1, 1)
  idx_ready(0, 0)
  gath(0, 0)
  idx_load(2, 0)
  idx_ready(1, 1)
  gath(1, 1)
  idx_load(3, 1)
  idx_ready(2, 2)
  gath(2, 2)

  step(0, 0, first=True)
  for j in (1, 2, 3):
    step(j, j)

  nquads = nchunk // 4

  @pl.loop(1, nquads)
  def _(i):
    step(4 * i, 0)
    step(4 * i + 1, 1)
    step(4 * i + 2, 2)
    step(4 * i + 3, 3)

  for j in range(4 * nquads, nchunk):
    step(j, j % 4)
  scat_wait(nchunk - 1, (nchunk - 1) % 4)


def _seg_scratch(tail):
  return [
      pltpu.VMEM_SHARED((NP, D), jnp.float32),
      pltpu.VMEM_SHARED((NP,), jnp.float32),
      [pltpu.VMEM((CHUNK,), jnp.int32)] * 2,
      [pltpu.VMEM((CHUNK,), jnp.int32)] * 2,
      [pltpu.VMEM((CHUNK,), jnp.int32)] * 4,
      [pltpu.VMEM((CHUNK,), jnp.int32)] * 4,
      [pltpu.VMEM((CHUNK, D), jnp.float32)] * 4,
      pltpu.VMEM((CHUNK,), jnp.float32),
      pltpu.VMEM((tail,), jnp.int32),
      pltpu.VMEM((tail,), jnp.int32),
      pltpu.VMEM((tail, D), jnp.float32),
      pltpu.VMEM((tail,), jnp.float32),
      [pltpu.SemaphoreType.DMA] * 4,
      [pltpu.SemaphoreType.DMA] * 4,
      [pltpu.SemaphoreType.DMA] * 2,
  ]


def _seg_sum_split_edges(table, edges_flat, z2, z1):
  ept = E // (NSC * NSUB)
  nchunk = ept // CHUNK
  tail = ept - nchunk * CHUNK

  @functools.partial(
      pl.kernel,
      out_type=(jax.ShapeDtypeStruct((NSC, NP, D), jnp.float32),
                jax.ShapeDtypeStruct((NSC, NP), jnp.float32)),
      mesh=_sc_mesh(),
      scratch_types=_seg_scratch(tail),
  )
  def k(table_hbm, ei_hbm, z2_hbm, z1_hbm, agg_hbm, cnt_hbm,
        acc_sh, cnt_sh, psrc_v, pdst_v, src_v, dst_v, rows_v, ones_v,
        tsrc_v, tdst_v, trows_v, tones_v, g_s, s_s, i_s):
    cid = lax.axis_index("c")
    sid = lax.axis_index("s")
    r0 = sid * RPS
    pltpu.sync_copy(z2_hbm.at[pl.ds(r0, RPS)], acc_sh.at[pl.ds(r0, RPS)])
    pltpu.sync_copy(z1_hbm.at[pl.ds(r0, RPS)], cnt_sh.at[pl.ds(r0, RPS)])
    for i in range(CHUNK // 16):
      ones_v[pl.ds(i * 16, 16)] = jnp.ones((16,), jnp.float32)
    for i in range(tail // 16):
      tones_v[pl.ds(i * 16, 16)] = jnp.ones((16,), jnp.float32)
    plsc.subcore_barrier()

    base = (cid * NSUB + sid) * ept

    def idx_load(j, p):
      off = base + j * CHUNK
      pltpu.async_copy(ei_hbm.at[pl.ds(off, CHUNK)], psrc_v[p], i_s[p])
      pltpu.async_copy(ei_hbm.at[pl.ds(E + off, CHUNK)], pdst_v[p], i_s[p])

    def idx_ready(j, b):
      p = b % 2
      off = base + j * CHUNK
      pltpu.make_async_copy(ei_hbm.at[pl.ds(off, CHUNK)], psrc_v[p],
                            i_s[p]).wait()
      pltpu.make_async_copy(ei_hbm.at[pl.ds(E + off, CHUNK)], pdst_v[p],
                            i_s[p]).wait()
      for i in range(CHUNK // 16):
        sl = pl.ds(i * 16, 16)
        src_v[b][sl] = psrc_v[p][sl]
        dst_v[b][sl] = pdst_v[p][sl]

    def gath(j, x):
      pltpu.async_copy(table_hbm.at[src_v[x]], rows_v[x], g_s[x])

    def gath_wait(j, x):
      pltpu.make_async_copy(table_hbm.at[src_v[x]], rows_v[x],
                            g_s[x]).wait()

    def scat(j, x):
      pltpu.async_copy(rows_v[x], acc_sh.at[dst_v[x]], s_s[x], add=True)
      pltpu.async_copy(ones_v, cnt_sh.at[dst_v[x]], s_s[x], add=True)

    def scat_wait(j, x):
      pltpu.make_async_copy(rows_v[x], acc_sh.at[dst_v[x]], s_s[x]).wait()
      pltpu.make_async_copy(ones_v, cnt_sh.at[dst_v[x]], s_s[x]).wait()

    _seg_pipeline(nchunk, gath, gath_wait, scat, scat_wait,
                  idx_load, idx_ready)

    if tail:
      toff = base + nchunk * CHUNK
      pltpu.sync_copy(ei_hbm.at[pl.ds(toff, tail)], tsrc_v)
      pltpu.sync_copy(ei_hbm.at[pl.ds(E + toff, tail)], tdst_v)
      pltpu.sync_copy(table_hbm.at[tsrc_v], trows_v)
      pltpu.sync_copy(trows_v, acc_sh.at[tdst_v], add=True)
      pltpu.sync_copy(tones_v, cnt_sh.at[tdst_v], add=True)

    plsc.subcore_barrier()
    pltpu.sync_copy(acc_sh.at[pl.ds(r0, RPS)], agg_hbm.at[cid, pl.ds(r0, RPS)])
    pltpu.sync_copy(cnt_sh.at[pl.ds(r0, RPS)], cnt_hbm.at[cid, pl.ds(r0, RPS)])

  return k(table, edges_flat, z2, z1)


def _seg_sum_two_tables(tables_flat, edges_flat, z2, z1):
  ept = E // NSUB
  nchunk = ept // CHUNK
  tail = ept - nchunk * CHUNK

  @functools.partial(
      pl.kernel,
      out_type=(jax.ShapeDtypeStruct((NSC, NP, D), jnp.float32),
                jax.ShapeDtypeStruct((NP,), jnp.float32)),
      mesh=_sc_mesh(),
      scratch_types=_seg_scratch(tail),
  )
  def k(tab_hbm, ei_hbm, z2_hbm, z1_hbm, agg_hbm, cnt_hbm,
        acc_sh, cnt_sh, psrc_v, pdst_v, src_v, dst_v, rows_v, ones_v,
        tsrc_v, tdst_v, trows_v, tones_v, g_s, s_s, i_s):
    cid = lax.axis_index("c")
    sid = lax.axis_index("s")
    r0 = sid * RPS
    row_off = cid * NP
    pltpu.sync_copy(z2_hbm.at[pl.ds(r0, RPS)], acc_sh.at[pl.ds(r0, RPS)])

    @pl.when(cid == 0)
    def _():
      pltpu.sync_copy(z1_hbm.at[pl.ds(r0, RPS)], cnt_sh.at[pl.ds(r0, RPS)])

    for i in range(CHUNK // 16):
      ones_v[pl.ds(i * 16, 16)] = jnp.ones((16,), jnp.float32)
    for i in range(tail // 16):
      tones_v[pl.ds(i * 16, 16)] = jnp.ones((16,), jnp.float32)
    plsc.subcore_barrier()

    base = sid * ept

    def idx_load(j, p):
      off = base + j * CHUNK
      pltpu.async_copy(ei_hbm.at[pl.ds(off, CHUNK)], psrc_v[p], i_s[p])
      pltpu.async_copy(ei_hbm.at[pl.ds(E + off, CHUNK)], pdst_v[p], i_s[p])

    def idx_ready(j, b):
      p = b % 2
      off = base + j * CHUNK
      pltpu.make_async_copy(ei_hbm.at[pl.ds(off, CHUNK)], psrc_v[p],
                            i_s[p]).wait()
      pltpu.make_async_copy(ei_hbm.at[pl.ds(E + off, CHUNK)], pdst_v[p],
                            i_s[p]).wait()
      for i in range(CHUNK // 16):
        sl = pl.ds(i * 16, 16)
        src_v[b][sl] = psrc_v[p][sl] + row_off
        dst_v[b][sl] = pdst_v[p][sl]

    def gath(j, x):
      pltpu.async_copy(tab_hbm.at[src_v[x]], rows_v[x], g_s[x])

    def gath_wait(j, x):
      pltpu.make_async_copy(tab_hbm.at[src_v[x]], rows_v[x],
                            g_s[x]).wait()

    def scat(j, x):
      pltpu.async_copy(rows_v[x], acc_sh.at[dst_v[x]], s_s[x], add=True)

      @pl.when(cid == 0)
      def _():
        pltpu.async_copy(ones_v, cnt_sh.at[dst_v[x]], s_s[x], add=True)

    def scat_wait(j, x):
      pltpu.make_async_copy(rows_v[x], acc_sh.at[dst_v[x]], s_s[x]).wait()

      @pl.when(cid == 0)
      def _():
        pltpu.make_async_copy(ones_v, cnt_sh.at[dst_v[x]], s_s[x]).wait()

    _seg_pipeline(nchunk, gath, gath_wait, scat, scat_wait,
                  idx_load, idx_ready)

    if tail:
      toff = base + nchunk * CHUNK
      pltpu.sync_copy(ei_hbm.at[pl.ds(toff, tail)], tsrc_v)
      pltpu.sync_copy(ei_hbm.at[pl.ds(E + toff, tail)], tdst_v)
      for i in range(tail // 16):
        sl = pl.ds(i * 16, 16)
        tsrc_v[sl] = tsrc_v[sl] + row_off
      pltpu.sync_copy(tab_hbm.at[tsrc_v], trows_v)
      pltpu.sync_copy(trows_v, acc_sh.at[tdst_v], add=True)

      @pl.when(cid == 0)
      def _():
        pltpu.sync_copy(tones_v, cnt_sh.at[tdst_v], add=True)

    plsc.subcore_barrier()
    pltpu.sync_copy(acc_sh.at[pl.ds(r0, RPS)], agg_hbm.at[cid, pl.ds(r0, RPS)])

    @pl.when(cid == 0)
    def _():
      pltpu.sync_copy(cnt_sh.at[pl.ds(r0, RPS)], cnt_hbm.at[pl.ds(r0, RPS)])

  return k(tables_flat, edges_flat, z2, z1)


def _hop1_combine(aggP, cntP, x_item, wl, wr, b, bu):
  def body(a_ref, c_ref, xi_ref, wl_ref, wr_ref, b_ref, bu_ref, o_ref):
    agg = a_ref[0] + a_ref[1]
    cnt = c_ref[0] + c_ref[1]
    inv = 1.0 / jnp.maximum(cnt, 1.0)
    mask = jnp.where(cnt > 0.0, 1.0, 0.0)
    z = (jnp.dot(agg, wl_ref[...], preferred_element_type=jnp.float32,
                 precision=lax.Precision.DEFAULT)
         * inv[:, None]
         + mask[:, None] * bu_ref[0]
         + jnp.dot(xi_ref[...], wr_ref[...],
                   preferred_element_type=jnp.float32,
                   precision=lax.Precision.DEFAULT)
         + b_ref[0])
    h = jnp.maximum(z, 0.0)
    o_ref[0] = h[:, :D]
    o_ref[1] = h[:, D:]

  return pl.pallas_call(
      body,
      grid=(NP // BR,),
      in_specs=[
          pl.BlockSpec((2, BR, D), lambda i: (0, i, 0)),
          pl.BlockSpec((2, BR), lambda i: (0, i)),
          pl.BlockSpec((BR, D), lambda i: (i, 0)),
          pl.BlockSpec((D, 2 * D), lambda i: (0, 0)),
          pl.BlockSpec((D, 2 * D), lambda i: (0, 0)),
          pl.BlockSpec((1, 2 * D), lambda i: (0, 0)),
          pl.BlockSpec((1, 2 * D), lambda i: (0, 0)),
      ],
      out_specs=pl.BlockSpec((2, BR, D), lambda i: (0, i, 0)),
      out_shape=jax.ShapeDtypeStruct((2, NP, D), jnp.float32),
  )(aggP, cntP, x_item, wl, wr, b, bu)


def _hop2_final(agg2, cnt2, x_user, wlA, wrA, bA, wlB, wrB, bB, v0, v1, c):
  def body(c_s, a_ref, c2_ref, xu_ref, wla, wra, ba, wlb, wrb, bb,
           v0_ref, v1_ref, o_ref):
    inv = 1.0 / jnp.maximum(c2_ref[...], 1.0)
    xu_b = xu_ref[...]
    zA = (jnp.dot(a_ref[0], wla[...], preferred_element_type=jnp.float32,
                  precision=lax.Precision.DEFAULT)
          * inv[:, None]
          + jnp.dot(xu_b, wra[...], preferred_element_type=jnp.float32,
                    precision=lax.Precision.DEFAULT)
          + ba[0])
    zB = (jnp.dot(a_ref[1], wlb[...], preferred_element_type=jnp.float32,
                  precision=lax.Precision.DEFAULT)
          * inv[:, None]
          + jnp.dot(xu_b, wrb[...], preferred_element_type=jnp.float32,
                    precision=lax.Precision.DEFAULT)
          + bb[0])
    hA = jnp.maximum(zA, 0.0)
    hB = jnp.maximum(zB, 0.0)
    y = (jnp.sum(hA * v0_ref[0], axis=1)
         + jnp.sum(hB * v1_ref[0], axis=1) + c_s[0])
    o_ref[0, 0, :] = y

  return pl.pallas_call(
      body,
      grid=(NP // BR,),
      in_specs=[
          pl.BlockSpec(memory_space=pltpu.SMEM),
          pl.BlockSpec((2, BR, D), lambda i: (0, i, 0)),
          pl.BlockSpec((BR,), lambda i: (i,)),
          pl.BlockSpec((BR, D), lambda i: (i, 0)),
          pl.BlockSpec((D, D), lambda i: (0, 0)),
          pl.BlockSpec((D, D), lambda i: (0, 0)),
          pl.BlockSpec((1, D), lambda i: (0, 0)),
          pl.BlockSpec((D, D), lambda i: (0, 0)),
          pl.BlockSpec((D, D), lambda i: (0, 0)),
          pl.BlockSpec((1, D), lambda i: (0, 0)),
          pl.BlockSpec((1, D), lambda i: (0, 0)),
          pl.BlockSpec((1, D), lambda i: (0, 0)),
      ],
      out_specs=pl.BlockSpec((1, 1, BR), lambda i: (i, 0, 0)),
      out_shape=jax.ShapeDtypeStruct((NP // BR, 1, BR), jnp.float32),
  )(c, agg2, cnt2, x_user, wlA, wrA, bA, wlB, wrB, bB, v0, v1)


def kernel(x_user, x_item, edge_u2i, edge_i2u, W_enc_u, b_enc_u, W_enc_i,
           b_enc_i, Wl00, bl00, Wr00, Wl01, bl01, Wr01, Wp0, bp0, Wl10,
           bl10, Wr10, Wl11, bl11, Wr11, Wp1, bp1, W_reg, b_reg):
  f32 = jnp.float32

  wl1 = jnp.concatenate([Wl00.T, Wl10.T], axis=1)
  wr1 = jnp.concatenate([Wr00.T, Wr10.T], axis=1)
  bl1 = jnp.concatenate([bl00, bl10])
  wlB = W_enc_u.T @ wl1
  buB = (b_enc_u @ wl1)[None, :]
  wrB = W_enc_i.T @ wr1
  bB = (b_enc_i @ wr1 + bl1)[None, :]
  wlCA = Wl01.T
  wrCA = W_enc_u.T @ Wr01.T
  bCA = (b_enc_u @ Wr01.T + bl01)[None, :]
  buCA = jnp.zeros((1, D), f32)
  wlCB = Wl11.T
  wrCB = W_enc_u.T @ Wr11.T
  bCB = (b_enc_u @ Wr11.T + bl11)[None, :]
  buCB = jnp.zeros((1, D), f32)
  wreg = W_reg[0]
  v0 = (Wp0.T @ wreg[:64])[None, :]
  v1 = (Wp1.T @ wreg[64:])[None, :]
  c = (jnp.dot(bp0, wreg[:64]) + jnp.dot(bp1, wreg[64:])
       + b_reg[0]).reshape(1).astype(f32)

  z2 = jnp.zeros((NP, D), f32)
  z1 = jnp.zeros((NP,), f32)

  agg1P, cnt1P = _seg_sum_split_edges(x_user, edge_u2i.reshape(2 * E),
                                      z2, z1)
  h01 = _hop1_combine(agg1P, cnt1P, x_item, wlB, wrB, bB, buB)
  agg2, cnt2 = _seg_sum_two_tables(h01.reshape(NSC * NP, D),
                                   edge_i2u.reshape(2 * E), z2, z1)
  y = _hop2_final(agg2, cnt2, x_user, wlCA, wrCA, bCA, wlCB, wrCB, bCB,
                  v0, v1, c)
  return y.reshape(NP)[:N]


# speedup vs baseline: 1.0258x; 1.0104x over previous
import functools

import jax
import jax.numpy as jnp
from jax import lax
from jax.experimental import pallas as pl
from jax.experimental.pallas import tpu as pltpu
from jax.experimental.pallas import tpu_sc as plsc

N = 10000
NP = 10240
D = 128
E = 320000
NSC = 2
NSUB = 16
CHUNK = 80
RPS = NP // NSUB
BR = 2048

_sc_mesh = functools.partial(
    plsc.VectorSubcoreMesh, core_axis_name="c", subcore_axis_name="s")


def _seg_pipeline(nchunk, gath, gath_wait, scat, scat_wait,
                  idx_load, idx_ready):
  def step(j, b, first=False):
    gath_wait(j, b)
    scat(j, b)
    if not first:
      scat_wait(j - 1, (b + 3) % 4)

    @pl.when(j + 3 < nchunk)
    def _():
      idx_ready(j + 3, (b + 3) % 4)
      gath(j + 3, (b + 3) % 4)

    @pl.when(j + 4 < nchunk)
    def _():
      idx_load(j + 4, b % 2)

  idx_load(0, 0)
  idx_load(1, 1)
  idx_ready(0, 0)
  gath(0, 0)
  idx_load(2, 0)
  idx_ready(1, 1)
  gath(1, 1)
  idx_load(3, 1)
  idx_ready(2, 2)
  gath(2, 2)

  step(0, 0, first=True)
  for j in (1, 2, 3):
    step(j, j)

  nquads = nchunk // 4

  @pl.loop(1, nquads)
  def _(i):
    step(4 * i, 0)
    step(4 * i + 1, 1)
    step(4 * i + 2, 2)
    step(4 * i + 3, 3)

  for j in range(4 * nquads, nchunk):
    step(j, j % 4)
  scat_wait(nchunk - 1, (nchunk - 1) % 4)


def _seg_scratch(tail):
  return [
      pltpu.VMEM_SHARED((NP, D), jnp.float32),
      pltpu.VMEM_SHARED((NP,), jnp.float32),
      [pltpu.VMEM((CHUNK,), jnp.int32)] * 2,
      [pltpu.VMEM((CHUNK,), jnp.int32)] * 2,
      [pltpu.VMEM((CHUNK,), jnp.int32)] * 4,
      [pltpu.VMEM((CHUNK,), jnp.int32)] * 4,
      [pltpu.VMEM((CHUNK, D), jnp.float32)] * 4,
      pltpu.VMEM((CHUNK,), jnp.float32),
      pltpu.VMEM((tail,), jnp.int32),
      pltpu.VMEM((tail,), jnp.int32),
      pltpu.VMEM((tail, D), jnp.float32),
      pltpu.VMEM((tail,), jnp.float32),
      [pltpu.SemaphoreType.DMA] * 4,
      [pltpu.SemaphoreType.DMA] * 4,
      [pltpu.SemaphoreType.DMA] * 2,
  ]


def _seg_sum_split_edges(table, edges_flat, z2, z1):
  ept = E // (NSC * NSUB)
  nchunk = ept // CHUNK
  tail = ept - nchunk * CHUNK

  @functools.partial(
      pl.kernel,
      out_type=(jax.ShapeDtypeStruct((NSC, NP, D), jnp.float32),
                jax.ShapeDtypeStruct((NSC, NP), jnp.float32)),
      mesh=_sc_mesh(),
      scratch_types=_seg_scratch(tail),
  )
  def k(table_hbm, ei_hbm, z2_hbm, z1_hbm, agg_hbm, cnt_hbm,
        acc_sh, cnt_sh, psrc_v, pdst_v, src_v, dst_v, rows_v, ones_v,
        tsrc_v, tdst_v, trows_v, tones_v, g_s, s_s, i_s):
    cid = lax.axis_index("c")
    sid = lax.axis_index("s")
    r0 = sid * RPS
    pltpu.sync_copy(z2_hbm.at[pl.ds(r0, RPS)], acc_sh.at[pl.ds(r0, RPS)])
    pltpu.sync_copy(z1_hbm.at[pl.ds(r0, RPS)], cnt_sh.at[pl.ds(r0, RPS)])
    for i in range(CHUNK // 16):
      ones_v[pl.ds(i * 16, 16)] = jnp.ones((16,), jnp.float32)
    for i in range(tail // 16):
      tones_v[pl.ds(i * 16, 16)] = jnp.ones((16,), jnp.float32)
    plsc.subcore_barrier()

    base = (cid * NSUB + sid) * ept

    def idx_load(j, p):
      off = base + j * CHUNK
      pltpu.async_copy(ei_hbm.at[pl.ds(off, CHUNK)], psrc_v[p], i_s[p])
      pltpu.async_copy(ei_hbm.at[pl.ds(E + off, CHUNK)], pdst_v[p], i_s[p])

    def idx_ready(j, b):
      p = b % 2
      off = base + j * CHUNK
      pltpu.make_async_copy(ei_hbm.at[pl.ds(off, CHUNK)], psrc_v[p],
                            i_s[p]).wait()
      pltpu.make_async_copy(ei_hbm.at[pl.ds(E + off, CHUNK)], pdst_v[p],
                            i_s[p]).wait()
      for i in range(CHUNK // 16):
        sl = pl.ds(i * 16, 16)
        src_v[b][sl] = psrc_v[p][sl]
        dst_v[b][sl] = pdst_v[p][sl]

    def gath(j, x):
      pltpu.async_copy(table_hbm.at[src_v[x]], rows_v[x], g_s[x])

    def gath_wait(j, x):
      pltpu.make_async_copy(table_hbm.at[src_v[x]], rows_v[x],
                            g_s[x]).wait()

    def scat(j, x):
      pltpu.async_copy(rows_v[x], acc_sh.at[dst_v[x]], s_s[x], add=True)
      pltpu.async_copy(ones_v, cnt_sh.at[dst_v[x]], s_s[x], add=True)

    def scat_wait(j, x):
      pltpu.make_async_copy(rows_v[x], acc_sh.at[dst_v[x]], s_s[x]).wait()
      pltpu.make_async_copy(ones_v, cnt_sh.at[dst_v[x]], s_s[x]).wait()

    _seg_pipeline(nchunk, gath, gath_wait, scat, scat_wait,
                  idx_load, idx_ready)

    if tail:
      toff = base + nchunk * CHUNK
      pltpu.sync_copy(ei_hbm.at[pl.ds(toff, tail)], tsrc_v)
      pltpu.sync_copy(ei_hbm.at[pl.ds(E + toff, tail)], tdst_v)
      pltpu.sync_copy(table_hbm.at[tsrc_v], trows_v)
      pltpu.sync_copy(trows_v, acc_sh.at[tdst_v], add=True)
      pltpu.sync_copy(tones_v, cnt_sh.at[tdst_v], add=True)

    plsc.subcore_barrier()
    pltpu.sync_copy(acc_sh.at[pl.ds(r0, RPS)], agg_hbm.at[cid, pl.ds(r0, RPS)])
    pltpu.sync_copy(cnt_sh.at[pl.ds(r0, RPS)], cnt_hbm.at[cid, pl.ds(r0, RPS)])

  return k(table, edges_flat, z2, z1)


def _seg_sum_two_tables(tables_flat, edges_flat, z2, z1):
  ept = E // NSUB
  nchunk = ept // CHUNK
  tail = ept - nchunk * CHUNK

  @functools.partial(
      pl.kernel,
      out_type=(jax.ShapeDtypeStruct((NSC, NP, D), jnp.float32),
                jax.ShapeDtypeStruct((NP,), jnp.float32)),
      mesh=_sc_mesh(),
      scratch_types=_seg_scratch(tail),
  )
  def k(tab_hbm, ei_hbm, z2_hbm, z1_hbm, agg_hbm, cnt_hbm,
        acc_sh, cnt_sh, psrc_v, pdst_v, src_v, dst_v, rows_v, ones_v,
        tsrc_v, tdst_v, trows_v, tones_v, g_s, s_s, i_s):
    cid = lax.axis_index("c")
    sid = lax.axis_index("s")
    r0 = sid * RPS
    row_off = cid * NP
    pltpu.sync_copy(z2_hbm.at[pl.ds(r0, RPS)], acc_sh.at[pl.ds(r0, RPS)])

    @pl.when(cid == 0)
    def _():
      pltpu.sync_copy(z1_hbm.at[pl.ds(r0, RPS)], cnt_sh.at[pl.ds(r0, RPS)])

    for i in range(CHUNK // 16):
      ones_v[pl.ds(i * 16, 16)] = jnp.ones((16,), jnp.float32)
    for i in range(tail // 16):
      tones_v[pl.ds(i * 16, 16)] = jnp.ones((16,), jnp.float32)
    plsc.subcore_barrier()

    base = sid * ept

    def idx_load(j, p):
      off = base + j * CHUNK
      pltpu.async_copy(ei_hbm.at[pl.ds(off, CHUNK)], psrc_v[p], i_s[p])
      pltpu.async_copy(ei_hbm.at[pl.ds(E + off, CHUNK)], pdst_v[p], i_s[p])

    def idx_ready(j, b):
      p = b % 2
      off = base + j * CHUNK
      pltpu.make_async_copy(ei_hbm.at[pl.ds(off, CHUNK)], psrc_v[p],
                            i_s[p]).wait()
      pltpu.make_async_copy(ei_hbm.at[pl.ds(E + off, CHUNK)], pdst_v[p],
                            i_s[p]).wait()
      for i in range(CHUNK // 16):
        sl = pl.ds(i * 16, 16)
        src_v[b][sl] = psrc_v[p][sl] + row_off
        dst_v[b][sl] = pdst_v[p][sl]

    def gath(j, x):
      pltpu.async_copy(tab_hbm.at[src_v[x]], rows_v[x], g_s[x])

    def gath_wait(j, x):
      pltpu.make_async_copy(tab_hbm.at[src_v[x]], rows_v[x],
                            g_s[x]).wait()

    def scat(j, x):
      pltpu.async_copy(rows_v[x], acc_sh.at[dst_v[x]], s_s[x], add=True)

      @pl.when(cid == 0)
      def _():
        pltpu.async_copy(ones_v, cnt_sh.at[dst_v[x]], s_s[x], add=True)

    def scat_wait(j, x):
      pltpu.make_async_copy(rows_v[x], acc_sh.at[dst_v[x]], s_s[x]).wait()

      @pl.when(cid == 0)
      def _():
        pltpu.make_async_copy(ones_v, cnt_sh.at[dst_v[x]], s_s[x]).wait()

    _seg_pipeline(nchunk, gath, gath_wait, scat, scat_wait,
                  idx_load, idx_ready)

    if tail:
      toff = base + nchunk * CHUNK
      pltpu.sync_copy(ei_hbm.at[pl.ds(toff, tail)], tsrc_v)
      pltpu.sync_copy(ei_hbm.at[pl.ds(E + toff, tail)], tdst_v)
      for i in range(tail // 16):
        sl = pl.ds(i * 16, 16)
        tsrc_v[sl] = tsrc_v[sl] + row_off
      pltpu.sync_copy(tab_hbm.at[tsrc_v], trows_v)
      pltpu.sync_copy(trows_v, acc_sh.at[tdst_v], add=True)

      @pl.when(cid == 0)
      def _():
        pltpu.sync_copy(tones_v, cnt_sh.at[tdst_v], add=True)

    plsc.subcore_barrier()
    pltpu.sync_copy(acc_sh.at[pl.ds(r0, RPS)], agg_hbm.at[cid, pl.ds(r0, RPS)])

    @pl.when(cid == 0)
    def _():
      pltpu.sync_copy(cnt_sh.at[pl.ds(r0, RPS)], cnt_hbm.at[pl.ds(r0, RPS)])

  return k(tables_flat, edges_flat, z2, z1)


def _hop1_combine(aggP, cntP, x_item, wl, wr, b, bu):
  def body(a_ref, c_ref, xi_ref, wl_ref, wr_ref, b_ref, bu_ref, o_ref):
    agg = a_ref[0] + a_ref[1]
    cnt = c_ref[0] + c_ref[1]
    inv = 1.0 / jnp.maximum(cnt, 1.0)
    mask = jnp.where(cnt > 0.0, 1.0, 0.0)
    z = (jnp.dot(agg, wl_ref[...], preferred_element_type=jnp.float32,
                 precision=lax.Precision.DEFAULT)
         * inv[:, None]
         + mask[:, None] * bu_ref[0]
         + jnp.dot(xi_ref[...], wr_ref[...],
                   preferred_element_type=jnp.float32,
                   precision=lax.Precision.DEFAULT)
         + b_ref[0])
    h = jnp.maximum(z, 0.0)
    o_ref[0] = h[:, :D]
    o_ref[1] = h[:, D:]

  return pl.pallas_call(
      body,
      grid=(NP // BR,),
      in_specs=[
          pl.BlockSpec((2, BR, D), lambda i: (0, i, 0)),
          pl.BlockSpec((2, BR), lambda i: (0, i)),
          pl.BlockSpec((BR, D), lambda i: (i, 0)),
          pl.BlockSpec((D, 2 * D), lambda i: (0, 0)),
          pl.BlockSpec((D, 2 * D), lambda i: (0, 0)),
          pl.BlockSpec((1, 2 * D), lambda i: (0, 0)),
          pl.BlockSpec((1, 2 * D), lambda i: (0, 0)),
      ],
      out_specs=pl.BlockSpec((2, BR, D), lambda i: (0, i, 0)),
      out_shape=jax.ShapeDtypeStruct((2, NP, D), jnp.float32),
  )(aggP, cntP, x_item, wl, wr, b, bu)


def _hop2_final(agg2, cnt2, x_user, wlA, wrA, bA, wlB, wrB, bB, v0, v1, c):
  def body(c_s, a_ref, c2_ref, xu_ref, wla, wra, ba, wlb, wrb, bb,
           v0_ref, v1_ref, o_ref):
    inv = 1.0 / jnp.maximum(c2_ref[...], 1.0)
    xu_b = xu_ref[...]
    zA = (jnp.dot(a_ref[0], wla[...], preferred_element_type=jnp.float32,
                  precision=lax.Precision.DEFAULT)
          * inv[:, None]
          + jnp.dot(xu_b, wra[...], preferred_element_type=jnp.float32,
                    precision=lax.Precision.DEFAULT)
          + ba[0])
    zB = (jnp.dot(a_ref[1], wlb[...], preferred_element_type=jnp.float32,
                  precision=lax.Precision.DEFAULT)
          * inv[:, None]
          + jnp.dot(xu_b, wrb[...], preferred_element_type=jnp.float32,
                    precision=lax.Precision.DEFAULT)
          + bb[0])
    hA = jnp.maximum(zA, 0.0)
    hB = jnp.maximum(zB, 0.0)
    y = (jnp.sum(hA * v0_ref[0], axis=1)
         + jnp.sum(hB * v1_ref[0], axis=1) + c_s[0])
    o_ref[0, 0, :] = y

  return pl.pallas_call(
      body,
      grid=(NP // BR,),
      in_specs=[
          pl.BlockSpec(memory_space=pltpu.SMEM),
          pl.BlockSpec((2, BR, D), lambda i: (0, i, 0)),
          pl.BlockSpec((BR,), lambda i: (i,)),
          pl.BlockSpec((BR, D), lambda i: (i, 0)),
          pl.BlockSpec((D, D), lambda i: (0, 0)),
          pl.BlockSpec((D, D), lambda i: (0, 0)),
          pl.BlockSpec((1, D), lambda i: (0, 0)),
          pl.BlockSpec((D, D), lambda i: (0, 0)),
          pl.BlockSpec((D, D), lambda i: (0, 0)),
          pl.BlockSpec((1, D), lambda i: (0, 0)),
          pl.BlockSpec((1, D), lambda i: (0, 0)),
          pl.BlockSpec((1, D), lambda i: (0, 0)),
      ],
      out_specs=pl.BlockSpec((1, 1, BR), lambda i: (i, 0, 0)),
      out_shape=jax.ShapeDtypeStruct((NP // BR, 1, BR), jnp.float32),
  )(c, agg2, cnt2, x_user, wlA, wrA, bA, wlB, wrB, bB, v0, v1)


def kernel(x_user, x_item, edge_u2i, edge_i2u, W_enc_u, b_enc_u, W_enc_i,
           b_enc_i, Wl00, bl00, Wr00, Wl01, bl01, Wr01, Wp0, bp0, Wl10,
           bl10, Wr10, Wl11, bl11, Wr11, Wp1, bp1, W_reg, b_reg):
  f32 = jnp.float32

  wl1 = jnp.concatenate([Wl00.T, Wl10.T], axis=1)
  wr1 = jnp.concatenate([Wr00.T, Wr10.T], axis=1)
  bl1 = jnp.concatenate([bl00, bl10])
  wlB = W_enc_u.T @ wl1
  buB = (b_enc_u @ wl1)[None, :]
  wrB = W_enc_i.T @ wr1
  bB = (b_enc_i @ wr1 + bl1)[None, :]
  wlCA = Wl01.T
  wrCA = W_enc_u.T @ Wr01.T
  bCA = (b_enc_u @ Wr01.T + bl01)[None, :]
  buCA = jnp.zeros((1, D), f32)
  wlCB = Wl11.T
  wrCB = W_enc_u.T @ Wr11.T
  bCB = (b_enc_u @ Wr11.T + bl11)[None, :]
  buCB = jnp.zeros((1, D), f32)
  wreg = W_reg[0]
  v0 = (Wp0.T @ wreg[:64])[None, :]
  v1 = (Wp1.T @ wreg[64:])[None, :]
  c = (jnp.dot(bp0, wreg[:64]) + jnp.dot(bp1, wreg[64:])
       + b_reg[0]).reshape(1).astype(f32)

  z2 = jnp.zeros((NP, D), f32)
  z1 = jnp.zeros((NP,), f32)

  agg1P, cnt1P = _seg_sum_split_edges(x_user, edge_u2i.reshape(2 * E),
                                      z2, z1)
  h01 = _hop1_combine(agg1P, cnt1P, x_item, wlB, wrB, bB, buB)
  agg2, cnt2 = _seg_sum_two_tables(h01.reshape(NSC * NP, D),
                                   edge_i2u.reshape(2 * E), z2, z1)
  y = _hop2_final(agg2, cnt2, x_user, wlCA, wrCA, bCA, wlCB, wrCB, bCB,
                  v0, v1, c)
  return y.reshape(NP)[:N]
